# Initial kernel scaffold; baseline (speedup 1.0000x reference)
import jax
import jax.numpy as jnp
from jax.experimental import pallas as pl


def kernel(x, token_table, pos_table):
    raise NotImplementedError("write your pallas kernel here")


# speedup vs baseline: 1.3923x; 1.3923x over previous
import jax
import jax.numpy as jnp
from jax import lax
from jax.experimental import pallas as pl
from jax.experimental.pallas import tpu as pltpu
from jax.experimental.pallas import tpu_sc as plsc
import functools

B = 4096
S = 200
D = 32

NC = 2
NS = 16
NW = NC * NS

ROWS = B * S
ROWS_PER_W = ROWS // NW
CHUNK = 800
NCHUNK = ROWS_PER_W // CHUNK
SUB = 80
NSUB = CHUNK // SUB

_mesh = plsc.VectorSubcoreMesh(core_axis_name="c", subcore_axis_name="s")


@functools.partial(
    pl.kernel,
    mesh=_mesh,
    compiler_params=pltpu.CompilerParams(use_tc_tiling_on_sc=False),
    out_type=jax.ShapeDtypeStruct((ROWS, D), jnp.float32),
    scratch_types=[
        pltpu.VMEM((NSUB, SUB), jnp.int32),
        pltpu.VMEM((CHUNK, D), jnp.float32),
        pltpu.VMEM((S, D), jnp.float32),
        pltpu.SemaphoreType.DMA,
    ],
)
def _embed_sc(x_hbm, tok_hbm, pos_hbm, out_hbm, idx_v, rows_v, pos_v, sem):
    wid = lax.axis_index("s") * NC + lax.axis_index("c")

    pltpu.sync_copy(pos_hbm, pos_v)

    def chunk_body(c, _):
        base = (wid * NCHUNK + c) * CHUNK
        pltpu.sync_copy(x_hbm.at[wid, c], idx_v)
        cps = [
            pltpu.async_copy(tok_hbm.at[idx_v.at[j]],
                             rows_v.at[pl.ds(j * SUB, SUB)], sem)
            for j in range(NSUB)
        ]
        for cp in cps:
            cp.wait()

        def row_body(i, _):
            for q in range(CHUNK // S):
                r = q * S + i
                for h in range(D // 16):
                    sl = pl.ds(h * 16, 16)
                    rows_v[r, sl] = rows_v[r, sl] + pos_v[i, sl]
            return 0

        lax.fori_loop(0, S, row_body, 0)
        pltpu.sync_copy(rows_v, out_hbm.at[pl.ds(base, CHUNK)])
        return 0

    lax.fori_loop(0, NCHUNK, chunk_body, 0)


def kernel(x, token_table, pos_table):
    x_r = x.reshape(NW, NCHUNK, NSUB, SUB).astype(jnp.int32)
    out = _embed_sc(x_r, token_table, pos_table)
    return out.reshape(B, S, D)


# speedup vs baseline: 1.4644x; 1.0518x over previous
import jax
import jax.numpy as jnp
from jax import lax
from jax.experimental import pallas as pl
from jax.experimental.pallas import tpu as pltpu
from jax.experimental.pallas import tpu_sc as plsc
import functools

B = 4096
S = 200
D = 32

NC = 2
NS = 16
NW = NC * NS

ROWS = B * S
ROWS_PER_W = ROWS // NW
CHUNK = 800
NCHUNK = ROWS_PER_W // CHUNK
SUB = 80
NSUB = CHUNK // SUB

_mesh = plsc.VectorSubcoreMesh(core_axis_name="c", subcore_axis_name="s")


@functools.partial(
    pl.kernel,
    mesh=_mesh,
    compiler_params=pltpu.CompilerParams(use_tc_tiling_on_sc=False),
    out_type=jax.ShapeDtypeStruct((ROWS, D), jnp.float32),
    scratch_types=[
        pltpu.VMEM((NSUB, SUB), jnp.int32),
        pltpu.VMEM((NSUB, SUB), jnp.int32),
        pltpu.VMEM((CHUNK, D), jnp.float32),
        pltpu.VMEM((CHUNK, D), jnp.float32),
        pltpu.VMEM((S, D), jnp.float32),
        pltpu.SemaphoreType.DMA,
        pltpu.SemaphoreType.DMA,
        pltpu.SemaphoreType.DMA,
        pltpu.SemaphoreType.DMA,
    ],
)
def _embed_sc(x_hbm, tok_hbm, pos_hbm, out_hbm,
              idx0, idx1, rows0, rows1, pos_v, g0, g1, o0, o1):
    wid = lax.axis_index("s") * NC + lax.axis_index("c")
    idx = (idx0, idx1)
    rows = (rows0, rows1)
    gs = (g0, g1)
    os_ = (o0, o1)

    pltpu.sync_copy(pos_hbm, pos_v)

    def prefetch(c, b):
        pltpu.sync_copy(x_hbm.at[wid, c], idx[b])
        for j in range(NSUB):
            pltpu.async_copy(tok_hbm.at[idx[b].at[j]],
                             rows[b].at[pl.ds(j * SUB, SUB)], gs[b])

    prefetch(0, 0)

    def outer(c2, _):
        for b in range(2):
            c = c2 * 2 + b
            nb = 1 - b

            @pl.when(c < NCHUNK - 1)
            def _():
                @pl.when(c >= 1)
                def _():
                    pltpu.make_async_copy(
                        rows[nb], out_hbm.at[pl.ds(0, CHUNK)], os_[nb]).wait()
                prefetch(c + 1, nb)

            pltpu.make_async_copy(
                out_hbm.at[pl.ds(0, CHUNK)], rows[b], gs[b]).wait()

            def row_body(i2, _):
                for k in range(8):
                    i = i2 * 8 + k
                    for h in range(D // 16):
                        sl = pl.ds(h * 16, 16)
                        pv = pos_v[i, sl]
                        for q in range(CHUNK // S):
                            r = q * S + i
                            rows[b][r, sl] = rows[b][r, sl] + pv
                return 0

            lax.fori_loop(0, S // 8, row_body, 0)
            pltpu.async_copy(
                rows[b],
                out_hbm.at[pl.ds((wid * NCHUNK + c) * CHUNK, CHUNK)], os_[b])
        return 0

    lax.fori_loop(0, NCHUNK // 2, outer, 0)
    pltpu.make_async_copy(rows[0], out_hbm.at[pl.ds(0, CHUNK)], os_[0]).wait()
    pltpu.make_async_copy(rows[1], out_hbm.at[pl.ds(0, CHUNK)], os_[1]).wait()


def kernel(x, token_table, pos_table):
    x_r = x.reshape(NW, NCHUNK, NSUB, SUB).astype(jnp.int32)
    out = _embed_sc(x_r, token_table, pos_table)
    return out.reshape(B, S, D)
